# baseline (speedup 1.0000x reference)
import functools

import jax
import jax.numpy as jnp
from jax import lax
from jax.experimental import pallas as pl
from jax.experimental.pallas import tpu as pltpu
from jax.experimental.pallas import tpu_sc as plsc

K0 = 10
KT = 110

NQ = 1024
NK = 100000
CK = 2048
KPAD = 100352
QB = 256

NW = 32
ROWS_PW = NQ // NW
HCH = KPAD // 2
NVPG = 8
GROUPS = HCH // (16 * NVPG)
NCHUNKS = 2 * ROWS_PW
OPAD = 112
ACAP = 8
BCAP = 16
NEG = -3.0e38


def _score_kernel(q_ref, k_ref, s_ref):
    j = pl.program_id(1)
    s = lax.dot_general(q_ref[...], k_ref[...], (((1,), (1,)), ((), ())),
                        preferred_element_type=jnp.float32)
    col = j * CK + lax.broadcasted_iota(jnp.int32, (QB, CK), 1)
    s_ref[...] = jnp.where(col < NK, s, jnp.float32(NEG))


def _shuf(x, k):
    idx = jnp.bitwise_xor(lax.iota(jnp.int32, 16), k)
    dnums = lax.GatherDimensionNumbers(
        offset_dims=(), collapsed_slice_dims=(0,), start_index_map=(0,))
    return lax.gather(x, idx[:, None], dnums, (1,),
                      mode=lax.GatherScatterMode.PROMISE_IN_BOUNDS)


def _allmax(x):
    for k in (8, 4, 2, 1):
        x = jnp.maximum(x, _shuf(x, k))
    return x[0]


def _allsum(x):
    for k in (8, 4, 2, 1):
        x = x + _shuf(x, k)
    return x[0]


def _cas(a, b):
    ak, av = a
    bk, bv = b
    m = ak <= bk
    lo = (jnp.where(m, ak, bk), jnp.where(m, av, bv))
    hi = (jnp.where(m, bk, ak), jnp.where(m, bv, av))
    return lo, hi


def _rev(p):
    return (lax.rev(p[0], (0,)), lax.rev(p[1], (0,)))


def _vsort(p):
    k, v = lax.sort((p[0], p[1]), dimension=0, num_keys=1)
    return k, v


def _bitonic(seq):
    n = len(seq)
    if n == 1:
        return [_vsort(seq[0])]
    half = n // 2
    lo, hi = [], []
    for i in range(half):
        l, h = _cas(seq[i], seq[i + half])
        lo.append(l)
        hi.append(h)
    return _bitonic(lo) + _bitonic(hi)


def _merge(a, b):
    br = [_rev(p) for p in reversed(b)]
    return _bitonic(a + br)


def _sort_list(ps):
    ps = [_vsort(p) for p in ps]
    width = 1
    while width < len(ps):
        out = []
        for i in range(0, len(ps), 2 * width):
            out.append(_merge(ps[i:i + width], ps[i + width:i + 2 * width]))
        ps = [p for grp in out for p in grp]
        width *= 2
    return ps


def _topk_kernel(scores, out_v, out_i, buf, av, ai, bv, bi, ov, oi, sem):
    wid = lax.axis_index("s") * 2 + lax.axis_index("c")
    base_row = wid * ROWS_PW
    negf = jnp.float32(NEG)

    for i in range(ACAP):
        av[pl.ds(i * 16, 16)] = jnp.full((16,), negf, jnp.float32)
        ai[pl.ds(i * 16, 16)] = jnp.zeros((16,), jnp.int32)
    for i in range(BCAP):
        bv[pl.ds(i * 16, 16)] = jnp.full((16,), -negf, jnp.float32)
        bi[pl.ds(i * 16, 16)] = jnp.zeros((16,), jnp.int32)

    def chunk_src(c):
        return scores.at[base_row + lax.shift_right_logical(c, 1),
                         pl.ds(lax.mul(lax.rem(c, 2), HCH), HCH)]

    def chunk_dst(c):
        return buf.at[pl.ds(lax.mul(lax.rem(c, 2), HCH), HCH)]

    def merge_flush():
        bpairs = [(-bv[pl.ds(i * 16, 16)], bi[pl.ds(i * 16, 16)])
                  for i in range(BCAP)]
        bs = _sort_list(bpairs)
        apairs = [(av[pl.ds(i * 16, 16)], ai[pl.ds(i * 16, 16)])
                  for i in range(ACAP)]
        negp = (jnp.full((16,), negf, jnp.float32), jnp.zeros((16,), jnp.int32))
        merged = _merge([negp] * (BCAP - ACAP) + apairs, bs)
        top = merged[-ACAP:]
        for i, (k, v) in enumerate(top):
            av[pl.ds(i * 16, 16)] = k
            ai[pl.ds(i * 16, 16)] = v
        for i in range(BCAP):
            bv[pl.ds(i * 16, 16)] = jnp.full((16,), -negf, jnp.float32)
        return top[(128 - KT) // 16][0][(128 - KT) % 16], top

    pltpu.make_async_copy(chunk_src(jnp.int32(0)), chunk_dst(jnp.int32(0)),
                          sem).start()

    def chunk_body(c, carry):
        thresh, cnt = carry
        parity = lax.rem(c, 2)
        off = lax.mul(parity, HCH)
        pltpu.make_async_copy(chunk_src(c), chunk_dst(c), sem).wait()

        @pl.when(c < NCHUNKS - 1)
        def _prefetch():
            pltpu.make_async_copy(chunk_src(c + 1), chunk_dst(c + 1),
                                  sem).start()

        def group_body(g, gcarry):
            thresh, cnt = gcarry
            gbase = off + g * (16 * NVPG)
            vs = [buf[pl.ds(gbase + k * 16, 16)] for k in range(NVPG)]
            gmax = vs[0]
            for k in range(1, NVPG):
                gmax = jnp.maximum(gmax, vs[k])
            hit = _allmax(gmax) > thresh

            def slow(op):
                thresh, cnt = op
                colbase = g * (16 * NVPG)
                for k in range(NVPG):
                    v = vs[k]
                    m = v > thresh
                    idxv = (off + colbase + k * 16) + lax.iota(jnp.int32, 16)
                    vm = jnp.where(m, -v, jnp.float32(-NEG))
                    sk, sv = lax.sort((vm, idxv), dimension=0, num_keys=1)
                    bv[pl.ds(cnt, 16)] = sk
                    bi[pl.ds(cnt, 16)] = sv
                    cnt = cnt + _allsum(m.astype(jnp.int32))

                def do_flush(_):
                    t, _top = merge_flush()
                    return t, jnp.int32(0)

                return lax.cond(cnt >= NVPG * 16,
                                do_flush, lambda _: (thresh, cnt), None)

            return lax.cond(hit, slow, lambda op: op, (thresh, cnt))

        thresh, cnt = lax.fori_loop(0, GROUPS, group_body, (thresh, cnt))

        def finalize(_):
            _t, top = merge_flush()
            for j in range(OPAD // 16):
                ov[pl.ds(j * 16, 16)] = lax.rev(top[ACAP - 1 - j][0], (0,))
                oi[pl.ds(j * 16, 16)] = lax.rev(top[ACAP - 1 - j][1], (0,))
            row = base_row + lax.shift_right_logical(c, 1)
            pltpu.sync_copy(ov, out_v.at[row])
            pltpu.sync_copy(oi, out_i.at[row])
            for i in range(ACAP):
                av[pl.ds(i * 16, 16)] = jnp.full((16,), negf, jnp.float32)
                ai[pl.ds(i * 16, 16)] = jnp.zeros((16,), jnp.int32)
            return jnp.float32(NEG), jnp.int32(0)

        return lax.cond(parity == 1, finalize, lambda op: op, (thresh, cnt))

    lax.fori_loop(0, NCHUNKS, chunk_body, (jnp.float32(NEG), jnp.int32(0)))


@functools.cache
def _topk_call():
    return pl.kernel(
        _topk_kernel,
        out_type=(jax.ShapeDtypeStruct((NQ, OPAD), jnp.float32),
                  jax.ShapeDtypeStruct((NQ, OPAD), jnp.int32)),
        mesh=plsc.VectorSubcoreMesh(core_axis_name="c",
                                    subcore_axis_name="s", num_cores=2),
        scratch_types=[
            pltpu.VMEM((2 * HCH,), jnp.float32),
            pltpu.VMEM((16 * ACAP,), jnp.float32),
            pltpu.VMEM((16 * ACAP,), jnp.int32),
            pltpu.VMEM((16 * BCAP,), jnp.float32),
            pltpu.VMEM((16 * BCAP,), jnp.int32),
            pltpu.VMEM((OPAD,), jnp.float32),
            pltpu.VMEM((OPAD,), jnp.int32),
            pltpu.SemaphoreType.DMA,
        ],
    )


def kernel(queries, keys):
    nq, d = queries.shape
    nk, _ = keys.shape
    keys_p = jnp.pad(keys, ((0, KPAD - nk), (0, 0)))
    grid = (nq // QB, KPAD // CK)
    scores = pl.pallas_call(
        _score_kernel,
        grid=grid,
        in_specs=[
            pl.BlockSpec((QB, d), lambda i, j: (i, 0)),
            pl.BlockSpec((CK, d), lambda i, j: (j, 0)),
        ],
        out_specs=pl.BlockSpec((QB, CK), lambda i, j: (i, j)),
        out_shape=jax.ShapeDtypeStruct((nq, KPAD), jnp.float32),
    )(queries, keys_p)
    vals, idx = _topk_call()(scores)
    return vals[:, K0:KT], idx[:, K0:KT]


def _r1_kernel(queries, keys):
    nq, d = queries.shape
    nk, _ = keys.shape
    keys_p = jnp.pad(keys, ((0, KPAD - nk), (0, 0)))
    grid = (nq // QB, KPAD // CK)
    scores = pl.pallas_call(
        _score_kernel,
        grid=grid,
        in_specs=[
            pl.BlockSpec((QB, d), lambda i, j: (i, 0)),
            pl.BlockSpec((CK, d), lambda i, j: (j, 0)),
        ],
        out_specs=pl.BlockSpec((QB, CK), lambda i, j: (i, j)),
        out_shape=jax.ShapeDtypeStruct((nq, KPAD), jnp.float32),
    )(queries, keys_p)
    vals, idx = jax.lax.top_k(scores[:, :NK], KT)
    return vals[:, K0:KT], idx[:, K0:KT]

kernel = _r1_kernel


# speedup vs baseline: 47.4834x; 47.4834x over previous
import functools

import jax
import jax.numpy as jnp
from jax import lax
from jax.experimental import pallas as pl
from jax.experimental.pallas import tpu as pltpu
from jax.experimental.pallas import tpu_sc as plsc

K0 = 10
KT = 110

NQ = 1024
NK = 100000
CK = 2048
KPAD = 100352
QB = 256

NW = 32
ROWS_PW = NQ // NW
NL1 = KPAD // 128
NL2 = NL1 // 8
NL2P = 104
NL3 = NL2P // 8
OPAD = 112
NEG = -3.0e38


def _score_kernel(q_ref, k_ref, s_ref):
    j = pl.program_id(1)
    s = lax.dot_general(q_ref[...], k_ref[...], (((1,), (1,)), ((), ())),
                        preferred_element_type=jnp.float32)
    col = j * CK + lax.broadcasted_iota(jnp.int32, (QB, CK), 1)
    s_ref[...] = jnp.where(col < NK, s, jnp.float32(NEG))


def _iota():
    return lax.iota(jnp.int32, 16)


def _gather16(x, idx):
    dnums = lax.GatherDimensionNumbers(
        offset_dims=(), collapsed_slice_dims=(0,), start_index_map=(0,))
    return lax.gather(x, idx[:, None], dnums, (1,),
                      mode=lax.GatherScatterMode.PROMISE_IN_BOUNDS)


def _shuf(x, k):
    return _gather16(x, jnp.bitwise_xor(_iota(), k))


def _lane_max(x):
    for k in (8, 4, 2, 1):
        x = jnp.maximum(x, _shuf(x, k))
    return x[0]


def _lane_min(x):
    for k in (8, 4, 2, 1):
        x = jnp.minimum(x, _shuf(x, k))
    return x[0]


def _topk_kernel(scores, out_v, out_i, buf, l1, l2, l3, ov, oi):
    wid = lax.axis_index("s") * 2 + lax.axis_index("c")
    base_row = wid * ROWS_PW
    bigi = jnp.int32(2 ** 30)

    for i in range(NL2, NL2P):
        l2[pl.ds(i * 16, 16)] = jnp.full((16,), NEG, jnp.float32)

    def eq_search(ref, base, n, target):
        cand = jnp.full((16,), bigi, jnp.int32)
        for k in range(n):
            v = ref[pl.ds(base + k * 16, 16)]
            cand = jnp.minimum(cand,
                               jnp.where(v == target, k * 16 + _iota(), bigi))
        return _lane_min(cand)

    def max8(ref, base):
        m = ref[pl.ds(base, 16)]
        for k in range(1, 8):
            m = jnp.maximum(m, ref[pl.ds(base + k * 16, 16)])
        return m

    def row_body(r, carry):
        row = base_row + r
        pltpu.sync_copy(scores.at[row], buf)

        def b1(g, c):
            l1[pl.ds(g * 16, 16)] = max8(buf, g * 128)
            return c

        def b2(h, c):
            l2[pl.ds(h * 16, 16)] = max8(l1, h * 128)
            return c

        def b3(q, c):
            l3[pl.ds(q * 16, 16)] = max8(l2, q * 128)
            return c

        lax.fori_loop(0, NL1, b1, 0)
        lax.fori_loop(0, NL2, b2, 0)
        lax.fori_loop(0, NL3, b3, 0)

        def extract(e, c):
            bv = l3[pl.ds(0, 16)]
            for k in range(1, NL3):
                bv = jnp.maximum(bv, l3[pl.ds(k * 16, 16)])
            vstar = _lane_max(bv)

            f3 = eq_search(l3, 0, NL3, vstar)
            q = lax.shift_right_logical(f3, 4)
            f2 = q * 128 + eq_search(l2, q * 128, 8, vstar)
            h = lax.shift_right_logical(f2, 4)
            f1 = h * 128 + eq_search(l1, h * 128, 8, vstar)
            g = lax.shift_right_logical(f1, 4)
            pos = g * 128 + eq_search(buf, g * 128, 8, vstar)

            vb = pos & ~jnp.int32(15)
            lane = pos & 15
            x = buf[pl.ds(vb, 16)]
            buf[pl.ds(vb, 16)] = jnp.where(_iota() == lane,
                                           jnp.float32(NEG), x)
            l1[pl.ds(g * 16, 16)] = max8(buf, g * 128)
            l2[pl.ds(h * 16, 16)] = max8(l1, h * 128)
            l3[pl.ds(q * 16, 16)] = max8(l2, q * 128)

            ob = e & ~jnp.int32(15)
            olane = e & 15
            ovv = ov[pl.ds(ob, 16)]
            ov[pl.ds(ob, 16)] = jnp.where(_iota() == olane, vstar, ovv)
            oii = oi[pl.ds(ob, 16)]
            oi[pl.ds(ob, 16)] = jnp.where(_iota() == olane, pos, oii)
            return c

        lax.fori_loop(0, OPAD, extract, 0)
        pltpu.sync_copy(ov, out_v.at[row])
        pltpu.sync_copy(oi, out_i.at[row])
        return carry

    lax.fori_loop(0, ROWS_PW, row_body, 0)


@functools.cache
def _topk_call():
    return pl.kernel(
        _topk_kernel,
        out_type=(jax.ShapeDtypeStruct((NQ, OPAD), jnp.float32),
                  jax.ShapeDtypeStruct((NQ, OPAD), jnp.int32)),
        mesh=plsc.VectorSubcoreMesh(core_axis_name="c",
                                    subcore_axis_name="s", num_cores=2),
        scratch_types=[
            pltpu.VMEM((KPAD,), jnp.float32),
            pltpu.VMEM((NL1 * 16,), jnp.float32),
            pltpu.VMEM((NL2P * 16,), jnp.float32),
            pltpu.VMEM((NL3 * 16,), jnp.float32),
            pltpu.VMEM((OPAD,), jnp.float32),
            pltpu.VMEM((OPAD,), jnp.int32),
        ],
    )


def kernel(queries, keys):
    nq, d = queries.shape
    nk, _ = keys.shape
    keys_p = jnp.pad(keys, ((0, KPAD - nk), (0, 0)))
    grid = (nq // QB, KPAD // CK)
    scores = pl.pallas_call(
        _score_kernel,
        grid=grid,
        in_specs=[
            pl.BlockSpec((QB, d), lambda i, j: (i, 0)),
            pl.BlockSpec((CK, d), lambda i, j: (j, 0)),
        ],
        out_specs=pl.BlockSpec((QB, CK), lambda i, j: (i, j)),
        out_shape=jax.ShapeDtypeStruct((nq, KPAD), jnp.float32),
    )(queries, keys_p)
    vals, idx = _topk_call()(scores)
    return vals[:, K0:KT], idx[:, K0:KT]
